# baseline (speedup 1.0000x reference)
import functools

import jax
import jax.numpy as jnp
import numpy as np
from jax import lax
from jax.experimental import pallas as pl
from jax.experimental.pallas import tpu as pltpu
from jax.experimental.pallas import tpu_sc as plsc

_D = 256
_K = 16
_ROW_BLK = 256
_TILE = 128
_NEG = np.float32(-1e30)


def _proj_body(side_ref, wk_ref, bk_ref, wv_ref, bv_ref, k_ref, v_ref):
    s = side_ref[...]
    k_ref[...] = jnp.dot(s, wk_ref[...], preferred_element_type=jnp.float32) + bk_ref[...]
    v_ref[...] = jnp.dot(s, wv_ref[...], preferred_element_type=jnp.float32) + bv_ref[...]


def _score_body(n_valid, n_pad, ego_ref, wq_ref, bq_ref, kmat_ref,
                s_ref, t_ref, tau_ref):
    rows = ego_ref.shape[0]
    ntiles = n_pad // _TILE
    q = jnp.dot(ego_ref[...], wq_ref[...], preferred_element_type=jnp.float32) + bq_ref[...]
    scale = np.float32(1.0 / np.sqrt(_D))
    s = lax.dot_general(
        q, kmat_ref[...], (((1,), (1,)), ((), ())),
        preferred_element_type=jnp.float32) * scale
    col = lax.broadcasted_iota(jnp.int32, (rows, n_pad), 1)
    s = jnp.where(col < n_valid, s, _NEG)
    s_ref[...] = s

    m3 = jnp.max(s.reshape(rows, ntiles, _TILE), axis=2)
    tcol = lax.broadcasted_iota(jnp.int32, (rows, ntiles), 1)
    tl = []
    m = None
    for _ in range(_K):
        m = jnp.max(m3, axis=1, keepdims=True)
        hit = m3 == m
        tidx = jnp.min(jnp.where(hit, tcol, ntiles), axis=1, keepdims=True)
        tl.append(tidx)
        m3 = jnp.where(tcol == tidx, _NEG, m3)
    t_ref[...] = jnp.concatenate(tl, axis=1)
    tau_ref[...] = m


def _sc_attend_body(n_pad, rows_per_w, num_cores,
                    s_hbm, t_hbm, tau_hbm, v_hbm, out_hbm,
                    t_v, tau_v, tiles_v, tidx_v, vidx_v, vrows_v, out_v, bc_v,
                    sem_t0, sem_t1, sem_v):
    ntiles = n_pad // _TILE
    wid = lax.axis_index("s") * num_cores + lax.axis_index("c")
    base = wid * rows_per_w
    pltpu.sync_copy(t_hbm.at[pl.ds(base * _K, rows_per_w * _K)], t_v)
    pltpu.sync_copy(tau_hbm.at[pl.ds(base, rows_per_w)], tau_v)
    iota16 = lax.iota(jnp.int32, 16)
    sems = (sem_t0, sem_t1)

    def issue_tiles(r_loc, buf):
        t_row = plsc.load_gather(t_v, [r_loc * _K + iota16])
        tidx_v[buf, pl.ds(0, 16)] = t_row + (base + r_loc) * ntiles
        pltpu.async_copy(s_hbm.at[tidx_v.at[buf]], tiles_v.at[buf], sems[buf])

    def wait_tiles(buf):
        pltpu.make_async_copy(s_hbm.at[tidx_v.at[buf]], tiles_v.at[buf],
                              sems[buf]).wait()

    def process(r_loc, buf, obuf):
        tau_s = plsc.load_gather(tau_v, [jnp.full((16,), r_loc, jnp.int32)])
        rv = jnp.full((16,), _NEG, jnp.float32)
        ri = jnp.zeros((16,), jnp.int32)
        for j in range(_K):
            tb = plsc.load_gather(t_v, [jnp.full((16,), r_loc * _K + j, jnp.int32)])
            for c in range(_TILE // 16):
                val = tiles_v[buf, j, pl.ds(c * 16, 16)]
                colid = tb * _TILE + (c * 16) + iota16
                pred = jnp.any(val >= tau_s)

                def _merge(op):
                    rv0, ri0, v0, c0 = op
                    sv, si = plsc.sort_key_val(v0, c0, descending=True)
                    rrev = lax.rev(rv0, (0,))
                    irev = lax.rev(ri0, (0,))
                    mk = sv >= rrev
                    nv = jnp.where(mk, sv, rrev)
                    ni = jnp.where(mk, si, irev)
                    nv, ni = plsc.sort_key_val(nv, ni, descending=True)
                    return nv, ni, v0, c0

                rv, ri, _, _ = lax.cond(pred, _merge, lambda op: op,
                                        (rv, ri, val, colid))

        m1 = jnp.full((16,), jnp.max(rv), jnp.float32)
        w = jnp.exp(rv - m1)
        z = jnp.full((16,), jnp.sum(w), jnp.float32)
        wn = w / z

        vidx_v[...] = ri
        pltpu.async_copy(v_hbm.at[vidx_v], vrows_v, sem_v).wait()

        bc_v[pl.ds(16, 16)] = wn
        accs = [jnp.zeros((16,), jnp.float32) for _ in range(_D // 16)]
        for j in range(_K):
            wb = plsc.load_gather(bc_v, [jnp.full((16,), 16 + j, jnp.int32)])
            for dch in range(_D // 16):
                accs[dch] = accs[dch] + wb * vrows_v[j, pl.ds(dch * 16, 16)]
        for dch in range(_D // 16):
            out_v[obuf, pl.ds(dch * 16, 16)] = accs[dch]

    issue_tiles(0, 0)

    def pair_body(r2, carry):
        r0 = 2 * r2
        r1 = r0 + 1
        rn = jnp.minimum(r0 + 2, rows_per_w - 1)
        issue_tiles(r1, 1)
        wait_tiles(0)
        process(r0, 0, 0)
        issue_tiles(rn, 0)
        wait_tiles(1)
        process(r1, 1, 1)
        pltpu.sync_copy(out_v, out_hbm.at[pl.ds(base + r0, 2)])
        return carry

    lax.fori_loop(0, rows_per_w // 2, pair_body, 0)
    wait_tiles(0)


def _run_hybrid(ego_emb, side_emb, Wq, bq, Wk, bk, Wv, bv, interpret=False):
    n, d = ego_emb.shape
    n_pad = ((n + _ROW_BLK - 1) // _ROW_BLK) * _ROW_BLK
    grid = n_pad // _ROW_BLK
    ntiles = n_pad // _TILE
    ego_p = jnp.pad(ego_emb, ((0, n_pad - n), (0, 0)))
    side_p = jnp.pad(side_emb, ((0, n_pad - n), (0, 0)))

    kmat, vmat = pl.pallas_call(
        _proj_body,
        grid=(grid,),
        in_specs=[
            pl.BlockSpec((_ROW_BLK, d), lambda i: (i, 0)),
            pl.BlockSpec((d, d), lambda i: (0, 0)),
            pl.BlockSpec((1, d), lambda i: (0, 0)),
            pl.BlockSpec((d, d), lambda i: (0, 0)),
            pl.BlockSpec((1, d), lambda i: (0, 0)),
        ],
        out_specs=[
            pl.BlockSpec((_ROW_BLK, d), lambda i: (i, 0)),
            pl.BlockSpec((_ROW_BLK, d), lambda i: (i, 0)),
        ],
        out_shape=[
            jax.ShapeDtypeStruct((n_pad, d), jnp.float32),
            jax.ShapeDtypeStruct((n_pad, d), jnp.float32),
        ],
        interpret=interpret,
    )(side_p, Wk, bk[None, :], Wv, bv[None, :])

    smat, tmat, tau = pl.pallas_call(
        functools.partial(_score_body, n, n_pad),
        grid=(grid,),
        in_specs=[
            pl.BlockSpec((_ROW_BLK, d), lambda i: (i, 0)),
            pl.BlockSpec((d, d), lambda i: (0, 0)),
            pl.BlockSpec((1, d), lambda i: (0, 0)),
            pl.BlockSpec((n_pad, d), lambda i: (0, 0)),
        ],
        out_specs=[
            pl.BlockSpec((_ROW_BLK, n_pad), lambda i: (i, 0)),
            pl.BlockSpec((_ROW_BLK, _K), lambda i: (i, 0)),
            pl.BlockSpec((_ROW_BLK, 1), lambda i: (i, 0)),
        ],
        out_shape=[
            jax.ShapeDtypeStruct((n_pad, n_pad), jnp.float32),
            jax.ShapeDtypeStruct((n_pad, _K), jnp.int32),
            jax.ShapeDtypeStruct((n_pad, 1), jnp.float32),
        ],
        interpret=interpret,
    )(ego_p, Wq, bq[None, :], kmat)

    try:
        info = plsc.get_sparse_core_info()
        num_cores, num_subcores = info.num_cores, info.num_subcores
    except Exception:
        num_cores, num_subcores = 2, 16
    num_workers = num_cores * num_subcores
    rows_per_w = n_pad // num_workers
    mesh = plsc.VectorSubcoreMesh(core_axis_name="c", subcore_axis_name="s",
                                  num_cores=num_cores, num_subcores=num_subcores)
    sc_fn = pl.kernel(
        functools.partial(_sc_attend_body, n_pad, rows_per_w, num_cores),
        mesh=mesh,
        compiler_params=pltpu.CompilerParams(needs_layout_passes=False),
        interpret=interpret,
        out_type=jax.ShapeDtypeStruct((n_pad, d), jnp.float32),
        scratch_types=[
            pltpu.VMEM((rows_per_w * _K,), jnp.int32),
            pltpu.VMEM((rows_per_w,), jnp.float32),
            pltpu.VMEM((2, _K, _TILE), jnp.float32),
            pltpu.VMEM((2, 16), jnp.int32),
            pltpu.VMEM((16,), jnp.int32),
            pltpu.VMEM((_K, d), jnp.float32),
            pltpu.VMEM((2, d), jnp.float32),
            pltpu.VMEM((32,), jnp.float32),
            pltpu.SemaphoreType.DMA,
            pltpu.SemaphoreType.DMA,
            pltpu.SemaphoreType.DMA,
        ],
    )
    out = sc_fn(smat.reshape(n_pad * ntiles, _TILE),
                tmat.reshape(n_pad * _K),
                tau.reshape(n_pad),
                vmat)
    return out[:n]


def kernel(ego_emb, side_emb, Wq, bq, Wk, bk, Wv, bv):
    return _run_hybrid(ego_emb, side_emb, Wq, bq, Wk, bk, Wv, bv)


# speedup vs baseline: 1.1690x; 1.1690x over previous
import functools

import jax
import jax.numpy as jnp
import numpy as np
from jax import lax
from jax.experimental import pallas as pl
from jax.experimental.pallas import tpu as pltpu
from jax.experimental.pallas import tpu_sc as plsc

_D = 256
_K = 16
_ROW_BLK = 256
_SELW = 128
_NEG = np.float32(-1e30)


def _proj_body(side_ref, wk_ref, bk_ref, wv_ref, bv_ref, k_ref, v_ref):
    s = side_ref[...]
    k_ref[...] = jnp.dot(s, wk_ref[...], preferred_element_type=jnp.float32) + bk_ref[...]
    v_ref[...] = jnp.dot(s, wv_ref[...], preferred_element_type=jnp.float32) + bv_ref[...]


def _score_body(n_valid, n_pad, ego_ref, wq_ref, bq_ref, kmat_ref,
                s_ref, t_ref):
    rows = ego_ref.shape[0]
    nsel = n_pad // _SELW
    q = jnp.dot(ego_ref[...], wq_ref[...], preferred_element_type=jnp.float32) + bq_ref[...]
    scale = np.float32(1.0 / np.sqrt(_D))
    s = lax.dot_general(
        q, kmat_ref[...], (((1,), (1,)), ((), ())),
        preferred_element_type=jnp.float32) * scale
    col = lax.broadcasted_iota(jnp.int32, (rows, n_pad), 1)
    s = jnp.where(col < n_valid, s, _NEG)
    s_ref[...] = s

    m3 = jnp.max(s.reshape(rows, nsel, _SELW), axis=2)
    tcol = lax.broadcasted_iota(jnp.int32, (rows, nsel), 1)
    tl = []
    for _ in range(_K):
        m = jnp.max(m3, axis=1, keepdims=True)
        hit = m3 == m
        tidx = jnp.min(jnp.where(hit, tcol, nsel), axis=1, keepdims=True)
        tl.append(tidx)
        m3 = jnp.where(tcol == tidx, _NEG, m3)
    t_ref[...] = jnp.concatenate(tl, axis=1)


def _sc_attend_body(n_pad, rows_per_w, num_cores,
                    s_hbm, t_hbm, v_hbm, out_hbm,
                    t_v, tiles_v, tidx_v, vidx_v, vrows_v, out_v, bc_v,
                    sem_t0, sem_t1, sem_v):
    nsel = n_pad // _SELW
    wid = lax.axis_index("s") * num_cores + lax.axis_index("c")
    base = wid * rows_per_w
    pltpu.sync_copy(t_hbm.at[pl.ds(base * _K, rows_per_w * _K)], t_v)
    iota16 = lax.iota(jnp.int32, 16)
    sems = (sem_t0, sem_t1)

    def issue_tiles(r_loc, buf):
        t_row = plsc.load_gather(t_v, [r_loc * _K + iota16])
        tidx_v[buf, pl.ds(0, 16)] = t_row + (base + r_loc) * nsel
        pltpu.async_copy(s_hbm.at[tidx_v.at[buf]], tiles_v.at[buf], sems[buf])

    def wait_tiles(buf):
        pltpu.make_async_copy(s_hbm.at[tidx_v.at[buf]], tiles_v.at[buf],
                              sems[buf]).wait()

    def process(r_loc, buf, obuf):
        rv = jnp.full((16,), _NEG, jnp.float32)
        ri = jnp.zeros((16,), jnp.int32)
        for j in range(_K):
            tb = plsc.load_gather(t_v, [jnp.full((16,), r_loc * _K + j, jnp.int32)])
            for c in range(_SELW // 16):
                val = tiles_v[buf, j, pl.ds(c * 16, 16)]
                colid = tb * _SELW + (c * 16) + iota16
                sv, si = plsc.sort_key_val(val, colid, descending=True)
                rrev = lax.rev(rv, (0,))
                irev = lax.rev(ri, (0,))
                mk = sv >= rrev
                nv = jnp.where(mk, sv, rrev)
                ni = jnp.where(mk, si, irev)
                rv, ri = plsc.sort_key_val(nv, ni, descending=True)

        m1 = jnp.full((16,), jnp.max(rv), jnp.float32)
        w = jnp.exp(rv - m1)
        z = jnp.full((16,), jnp.sum(w), jnp.float32)
        wn = w / z

        vidx_v[...] = ri
        pltpu.async_copy(v_hbm.at[vidx_v], vrows_v, sem_v).wait()

        bc_v[pl.ds(16, 16)] = wn
        accs = [jnp.zeros((16,), jnp.float32) for _ in range(_D // 16)]
        for j in range(_K):
            wb = plsc.load_gather(bc_v, [jnp.full((16,), 16 + j, jnp.int32)])
            for dch in range(_D // 16):
                accs[dch] = accs[dch] + wb * vrows_v[j, pl.ds(dch * 16, 16)]
        for dch in range(_D // 16):
            out_v[obuf, pl.ds(dch * 16, 16)] = accs[dch]

    issue_tiles(0, 0)

    def pair_body(r2, carry):
        r0 = 2 * r2
        r1 = r0 + 1
        rn = jnp.minimum(r0 + 2, rows_per_w - 1)
        issue_tiles(r1, 1)
        wait_tiles(0)
        process(r0, 0, 0)
        issue_tiles(rn, 0)
        wait_tiles(1)
        process(r1, 1, 1)
        pltpu.sync_copy(out_v, out_hbm.at[pl.ds(base + r0, 2)])
        return carry

    lax.fori_loop(0, rows_per_w // 2, pair_body, 0)
    wait_tiles(0)


def _run_hybrid(ego_emb, side_emb, Wq, bq, Wk, bk, Wv, bv, interpret=False):
    n, d = ego_emb.shape
    n_pad = ((n + _ROW_BLK - 1) // _ROW_BLK) * _ROW_BLK
    grid = n_pad // _ROW_BLK
    nsel = n_pad // _SELW
    ego_p = jnp.pad(ego_emb, ((0, n_pad - n), (0, 0)))
    side_p = jnp.pad(side_emb, ((0, n_pad - n), (0, 0)))

    kmat, vmat = pl.pallas_call(
        _proj_body,
        grid=(grid,),
        in_specs=[
            pl.BlockSpec((_ROW_BLK, d), lambda i: (i, 0)),
            pl.BlockSpec((d, d), lambda i: (0, 0)),
            pl.BlockSpec((1, d), lambda i: (0, 0)),
            pl.BlockSpec((d, d), lambda i: (0, 0)),
            pl.BlockSpec((1, d), lambda i: (0, 0)),
        ],
        out_specs=[
            pl.BlockSpec((_ROW_BLK, d), lambda i: (i, 0)),
            pl.BlockSpec((_ROW_BLK, d), lambda i: (i, 0)),
        ],
        out_shape=[
            jax.ShapeDtypeStruct((n_pad, d), jnp.float32),
            jax.ShapeDtypeStruct((n_pad, d), jnp.float32),
        ],
        interpret=interpret,
    )(side_p, Wk, bk[None, :], Wv, bv[None, :])

    smat, tmat = pl.pallas_call(
        functools.partial(_score_body, n, n_pad),
        grid=(grid,),
        in_specs=[
            pl.BlockSpec((_ROW_BLK, d), lambda i: (i, 0)),
            pl.BlockSpec((d, d), lambda i: (0, 0)),
            pl.BlockSpec((1, d), lambda i: (0, 0)),
            pl.BlockSpec((n_pad, d), lambda i: (0, 0)),
        ],
        out_specs=[
            pl.BlockSpec((_ROW_BLK, n_pad), lambda i: (i, 0)),
            pl.BlockSpec((_ROW_BLK, _K), lambda i: (i, 0)),
        ],
        out_shape=[
            jax.ShapeDtypeStruct((n_pad, n_pad), jnp.float32),
            jax.ShapeDtypeStruct((n_pad, _K), jnp.int32),
        ],
        interpret=interpret,
    )(ego_p, Wq, bq[None, :], kmat)

    try:
        info = plsc.get_sparse_core_info()
        num_cores, num_subcores = info.num_cores, info.num_subcores
    except Exception:
        num_cores, num_subcores = 2, 16
    num_workers = num_cores * num_subcores
    rows_per_w = n_pad // num_workers
    mesh = plsc.VectorSubcoreMesh(core_axis_name="c", subcore_axis_name="s",
                                  num_cores=num_cores, num_subcores=num_subcores)
    sc_fn = pl.kernel(
        functools.partial(_sc_attend_body, n_pad, rows_per_w, num_cores),
        mesh=mesh,
        compiler_params=pltpu.CompilerParams(needs_layout_passes=False),
        interpret=interpret,
        out_type=jax.ShapeDtypeStruct((n_pad, d), jnp.float32),
        scratch_types=[
            pltpu.VMEM((rows_per_w * _K,), jnp.int32),
            pltpu.VMEM((2, _K, _SELW), jnp.float32),
            pltpu.VMEM((2, 16), jnp.int32),
            pltpu.VMEM((16,), jnp.int32),
            pltpu.VMEM((_K, d), jnp.float32),
            pltpu.VMEM((2, d), jnp.float32),
            pltpu.VMEM((32,), jnp.float32),
            pltpu.SemaphoreType.DMA,
            pltpu.SemaphoreType.DMA,
            pltpu.SemaphoreType.DMA,
        ],
    )
    out = sc_fn(smat.reshape(n_pad * nsel, _SELW),
                tmat.reshape(n_pad * _K),
                vmat)
    return out[:n]


def kernel(ego_emb, side_emb, Wq, bq, Wk, bk, Wv, bv):
    return _run_hybrid(ego_emb, side_emb, Wq, bq, Wk, bk, Wv, bv)


# speedup vs baseline: 1.5587x; 1.3333x over previous
import functools

import jax
import jax.numpy as jnp
import numpy as np
from jax import lax
from jax.experimental import pallas as pl
from jax.experimental.pallas import tpu as pltpu
from jax.experimental.pallas import tpu_sc as plsc

_D = 256
_K = 16
_ROW_BLK = 256
_SELW = 128
_NEG = np.float32(-1e30)


def _proj_body(side_ref, wk_ref, bk_ref, wv_ref, bv_ref, k_ref, v_ref):
    s = side_ref[...]
    k_ref[...] = jnp.dot(s, wk_ref[...], preferred_element_type=jnp.float32) + bk_ref[...]
    v_ref[...] = jnp.dot(s, wv_ref[...], preferred_element_type=jnp.float32) + bv_ref[...]


def _score_body(n_valid, n_pad, ego_ref, wq_ref, bq_ref, kmat_ref,
                s_ref, t_ref):
    rows = ego_ref.shape[0]
    nsel = n_pad // _SELW
    q = jnp.dot(ego_ref[...], wq_ref[...], preferred_element_type=jnp.float32) + bq_ref[...]
    scale = np.float32(1.0 / np.sqrt(_D))
    s = lax.dot_general(
        q, kmat_ref[...], (((1,), (1,)), ((), ())),
        preferred_element_type=jnp.float32) * scale
    col = lax.broadcasted_iota(jnp.int32, (rows, n_pad), 1)
    s = jnp.where(col < n_valid, s, _NEG)
    s_ref[...] = s

    m3 = jnp.max(s.reshape(rows, nsel, _SELW), axis=2)
    tcol = lax.broadcasted_iota(jnp.int32, (rows, nsel), 1)
    tl = []
    for _ in range(_K):
        m = jnp.max(m3, axis=1, keepdims=True)
        hit = m3 == m
        tidx = jnp.min(jnp.where(hit, tcol, nsel), axis=1, keepdims=True)
        tl.append(tidx)
        m3 = jnp.where(tcol == tidx, _NEG, m3)
    t_ref[...] = jnp.concatenate(tl, axis=1)


def _sc_attend_body(n_pad, rows_per_w, num_cores,
                    s_hbm, t_hbm, v_hbm, out_hbm,
                    t_v, tiles_v, tidx_v, vidx_v, vrows_v, out_v, bc_v,
                    sem_t0, sem_t1, sem_v):
    nsel = n_pad // _SELW
    wid = lax.axis_index("s") * num_cores + lax.axis_index("c")
    base = wid * rows_per_w
    pltpu.sync_copy(t_hbm.at[pl.ds(base * _K, rows_per_w * _K)], t_v)
    iota16 = lax.iota(jnp.int32, 16)
    sems = (sem_t0, sem_t1)

    def issue_tiles(r_loc, buf):
        t_row = plsc.load_gather(t_v, [r_loc * _K + iota16])
        tidx_v[buf, pl.ds(0, 16)] = t_row + (base + r_loc) * nsel
        pltpu.async_copy(s_hbm.at[tidx_v.at[buf]], tiles_v.at[buf], sems[buf])

    def wait_tiles(buf):
        pltpu.make_async_copy(s_hbm.at[tidx_v.at[buf]], tiles_v.at[buf],
                              sems[buf]).wait()

    def process(r_loc, buf, obuf):
        rv = jnp.full((16,), _NEG, jnp.float32)
        ri = jnp.zeros((16,), jnp.int32)
        for j in range(_K):
            tb = plsc.load_gather(t_v, [jnp.full((16,), r_loc * _K + j, jnp.int32)])
            for c in range(_SELW // 16):
                val = tiles_v[buf, j, pl.ds(c * 16, 16)]
                colid = tb * _SELW + (c * 16) + iota16
                sv, si = plsc.sort_key_val(val, colid, descending=True)
                rrev = lax.rev(rv, (0,))
                irev = lax.rev(ri, (0,))
                mk = sv >= rrev
                nv = jnp.where(mk, sv, rrev)
                ni = jnp.where(mk, si, irev)
                rv, ri = plsc.sort_key_val(nv, ni, descending=True)

        m1 = jnp.full((16,), jnp.max(rv), jnp.float32)
        w = jnp.exp(rv - m1)
        z = jnp.full((16,), jnp.sum(w), jnp.float32)
        wn = w / z

        vidx_v[...] = ri
        pltpu.async_copy(v_hbm.at[vidx_v], vrows_v, sem_v).wait()

        bc_v[pl.ds(16, 16)] = wn
        accs = [jnp.zeros((16,), jnp.float32) for _ in range(_D // 16)]
        for j in range(_K):
            wb = plsc.load_gather(bc_v, [jnp.full((16,), 16 + j, jnp.int32)])
            for dch in range(_D // 16):
                accs[dch] = accs[dch] + wb * vrows_v[j, pl.ds(dch * 16, 16)]
        for dch in range(_D // 16):
            out_v[obuf, pl.ds(dch * 16, 16)] = accs[dch]

    issue_tiles(0, 0)

    def pair_body(r2, carry):
        r0 = 2 * r2
        r1 = r0 + 1
        rn = jnp.minimum(r0 + 2, rows_per_w - 1)
        issue_tiles(r1, 1)
        wait_tiles(0)
        process(r0, 0, 0)
        issue_tiles(rn, 0)
        wait_tiles(1)
        process(r1, 1, 1)
        pltpu.sync_copy(out_v, out_hbm.at[pl.ds(base + r0, 2)])
        return carry

    lax.fori_loop(0, rows_per_w // 2, pair_body, 0)
    wait_tiles(0)


def _run_hybrid(ego_emb, side_emb, Wq, bq, Wk, bk, Wv, bv, interpret=False):
    n, d = ego_emb.shape
    n_pad = ((n + _ROW_BLK - 1) // _ROW_BLK) * _ROW_BLK
    grid = n_pad // _ROW_BLK
    nsel = n_pad // _SELW
    ego_p = jnp.pad(ego_emb, ((0, n_pad - n), (0, 0)))
    side_p = jnp.pad(side_emb, ((0, n_pad - n), (0, 0)))

    kmat, vmat = pl.pallas_call(
        _proj_body,
        grid=(grid,),
        in_specs=[
            pl.BlockSpec((_ROW_BLK, d), lambda i: (i, 0)),
            pl.BlockSpec((d, d), lambda i: (0, 0)),
            pl.BlockSpec((1, d), lambda i: (0, 0)),
            pl.BlockSpec((d, d), lambda i: (0, 0)),
            pl.BlockSpec((1, d), lambda i: (0, 0)),
        ],
        out_specs=[
            pl.BlockSpec((_ROW_BLK, d), lambda i: (i, 0)),
            pl.BlockSpec((_ROW_BLK, d), lambda i: (i, 0)),
        ],
        out_shape=[
            jax.ShapeDtypeStruct((n_pad, d), jnp.float32),
            jax.ShapeDtypeStruct((n_pad, d), jnp.float32),
        ],
        interpret=interpret,
    )(side_p, Wk, bk[None, :], Wv, bv[None, :])

    try:
        info = plsc.get_sparse_core_info()
        num_cores, num_subcores = info.num_cores, info.num_subcores
    except Exception:
        num_cores, num_subcores = 2, 16
    num_workers = num_cores * num_subcores

    nh = 4
    rows_h = n_pad // nh
    rows_per_w = rows_h // num_workers
    mesh = plsc.VectorSubcoreMesh(core_axis_name="c", subcore_axis_name="s",
                                  num_cores=num_cores, num_subcores=num_subcores)
    sc_fn = pl.kernel(
        functools.partial(_sc_attend_body, n_pad, rows_per_w, num_cores),
        mesh=mesh,
        compiler_params=pltpu.CompilerParams(needs_layout_passes=False),
        interpret=interpret,
        out_type=jax.ShapeDtypeStruct((rows_h, d), jnp.float32),
        scratch_types=[
            pltpu.VMEM((rows_per_w * _K,), jnp.int32),
            pltpu.VMEM((2, _K, _SELW), jnp.float32),
            pltpu.VMEM((2, 16), jnp.int32),
            pltpu.VMEM((16,), jnp.int32),
            pltpu.VMEM((_K, d), jnp.float32),
            pltpu.VMEM((2, d), jnp.float32),
            pltpu.VMEM((32,), jnp.float32),
            pltpu.SemaphoreType.DMA,
            pltpu.SemaphoreType.DMA,
            pltpu.SemaphoreType.DMA,
        ],
    )

    outs = []
    for h in range(nh):
        ego_h = jax.lax.slice_in_dim(ego_p, h * rows_h, (h + 1) * rows_h, axis=0)
        smat_h, tmat_h = pl.pallas_call(
            functools.partial(_score_body, n, n_pad),
            grid=(rows_h // _ROW_BLK,),
            in_specs=[
                pl.BlockSpec((_ROW_BLK, d), lambda i: (i, 0)),
                pl.BlockSpec((d, d), lambda i: (0, 0)),
                pl.BlockSpec((1, d), lambda i: (0, 0)),
                pl.BlockSpec((n_pad, d), lambda i: (0, 0)),
            ],
            out_specs=[
                pl.BlockSpec((_ROW_BLK, n_pad), lambda i: (i, 0)),
                pl.BlockSpec((_ROW_BLK, _K), lambda i: (i, 0)),
            ],
            out_shape=[
                jax.ShapeDtypeStruct((rows_h, n_pad), jnp.float32),
                jax.ShapeDtypeStruct((rows_h, _K), jnp.int32),
            ],
            interpret=interpret,
        )(ego_h, Wq, bq[None, :], kmat)
        outs.append(sc_fn(smat_h.reshape(rows_h * nsel, _SELW),
                          tmat_h.reshape(rows_h * _K),
                          vmat))
    out = jnp.concatenate(outs, axis=0)
    return out[:n]


def kernel(ego_emb, side_emb, Wq, bq, Wk, bk, Wv, bv):
    return _run_hybrid(ego_emb, side_emb, Wq, bq, Wk, bk, Wv, bv)


# speedup vs baseline: 1.5994x; 1.0261x over previous
import functools

import jax
import jax.numpy as jnp
import numpy as np
from jax import lax
from jax.experimental import pallas as pl
from jax.experimental.pallas import tpu as pltpu
from jax.experimental.pallas import tpu_sc as plsc

_D = 256
_K = 16
_ROW_BLK = 256
_SELW = 128
_NEG = np.float32(-1e30)


def _proj_body(side_ref, wk_ref, bk_ref, wv_ref, bv_ref, k_ref, v_ref):
    s = side_ref[...]
    k_ref[...] = jnp.dot(s, wk_ref[...], preferred_element_type=jnp.float32) + bk_ref[...]
    v_ref[...] = jnp.dot(s, wv_ref[...], preferred_element_type=jnp.float32) + bv_ref[...]


def _score_body(n_valid, n_pad, ego_ref, wq_ref, bq_ref, kmat_ref,
                s_ref, t_ref):
    rows = ego_ref.shape[0]
    nsel = n_pad // _SELW
    q = jnp.dot(ego_ref[...], wq_ref[...], preferred_element_type=jnp.float32) + bq_ref[...]
    scale = np.float32(1.0 / np.sqrt(_D))
    s = lax.dot_general(
        q, kmat_ref[...], (((1,), (1,)), ((), ())),
        preferred_element_type=jnp.float32) * scale
    col = lax.broadcasted_iota(jnp.int32, (rows, n_pad), 1)
    s = jnp.where(col < n_valid, s, _NEG)
    s_ref[...] = s

    m3 = jnp.max(s.reshape(rows, nsel, _SELW), axis=2)
    tcol = lax.broadcasted_iota(jnp.int32, (rows, nsel), 1)
    tl = []
    for _ in range(_K):
        m = jnp.max(m3, axis=1, keepdims=True)
        hit = m3 == m
        tidx = jnp.min(jnp.where(hit, tcol, nsel), axis=1, keepdims=True)
        tl.append(tidx)
        m3 = jnp.where(tcol == tidx, _NEG, m3)
    t_ref[...] = jnp.concatenate(tl, axis=1)


def _sc_attend_body(n_pad, rows_per_w, num_cores,
                    s_hbm, t_hbm, v_hbm, out_hbm,
                    t_v, tiles_v, tidx_v, vidx_v, vrows_v, out_v, bc_v,
                    sem_t0, sem_t1, sem_v):
    nsel = n_pad // _SELW
    wid = lax.axis_index("s") * num_cores + lax.axis_index("c")
    base = wid * rows_per_w
    pltpu.sync_copy(t_hbm.at[pl.ds(base * _K, rows_per_w * _K)], t_v)
    iota16 = lax.iota(jnp.int32, 16)
    sems = (sem_t0, sem_t1)

    def issue_tiles(r_loc, buf):
        t_row = plsc.load_gather(t_v, [r_loc * _K + iota16])
        tidx_v[buf, pl.ds(0, 16)] = t_row + (base + r_loc) * nsel
        pltpu.async_copy(s_hbm.at[tidx_v.at[buf]], tiles_v.at[buf], sems[buf])

    def wait_tiles(buf):
        pltpu.make_async_copy(s_hbm.at[tidx_v.at[buf]], tiles_v.at[buf],
                              sems[buf]).wait()

    def process(r_loc, buf, obuf):
        rv = jnp.full((16,), _NEG, jnp.float32)
        ri = jnp.zeros((16,), jnp.int32)
        for j in range(_K):
            tb = plsc.load_gather(t_v, [jnp.full((16,), r_loc * _K + j, jnp.int32)])
            for c in range(_SELW // 16):
                val = tiles_v[buf, j, pl.ds(c * 16, 16)]
                colid = tb * _SELW + (c * 16) + iota16
                sv, si = plsc.sort_key_val(val, colid, descending=True)
                rrev = lax.rev(rv, (0,))
                irev = lax.rev(ri, (0,))
                mk = sv >= rrev
                nv = jnp.where(mk, sv, rrev)
                ni = jnp.where(mk, si, irev)
                rv, ri = plsc.sort_key_val(nv, ni, descending=True)

        m1 = jnp.full((16,), jnp.max(rv), jnp.float32)
        w = jnp.exp(rv - m1)
        z = jnp.full((16,), jnp.sum(w), jnp.float32)
        wn = w / z

        vidx_v[...] = ri
        pltpu.async_copy(v_hbm.at[vidx_v], vrows_v, sem_v).wait()

        bc_v[pl.ds(16, 16)] = wn
        accs = [jnp.zeros((16,), jnp.float32) for _ in range(_D // 16)]
        for j in range(_K):
            wb = plsc.load_gather(bc_v, [jnp.full((16,), 16 + j, jnp.int32)])
            for dch in range(_D // 16):
                accs[dch] = accs[dch] + wb * vrows_v[j, pl.ds(dch * 16, 16)]
        for dch in range(_D // 16):
            out_v[obuf, pl.ds(dch * 16, 16)] = accs[dch]

    issue_tiles(0, 0)

    def pair_body(r2, carry):
        r0 = 2 * r2
        r1 = r0 + 1
        rn = jnp.minimum(r0 + 2, rows_per_w - 1)
        issue_tiles(r1, 1)
        wait_tiles(0)
        process(r0, 0, 0)
        issue_tiles(rn, 0)
        wait_tiles(1)
        process(r1, 1, 1)
        pltpu.sync_copy(out_v, out_hbm.at[pl.ds(base + r0, 2)])
        return carry

    lax.fori_loop(0, rows_per_w // 2, pair_body, 0)
    wait_tiles(0)


def _run_hybrid(ego_emb, side_emb, Wq, bq, Wk, bk, Wv, bv, interpret=False):
    n, d = ego_emb.shape
    n_pad = ((n + _ROW_BLK - 1) // _ROW_BLK) * _ROW_BLK
    grid = n_pad // _ROW_BLK
    nsel = n_pad // _SELW
    ego_p = jnp.pad(ego_emb, ((0, n_pad - n), (0, 0)))
    side_p = jnp.pad(side_emb, ((0, n_pad - n), (0, 0)))

    kmat, vmat = pl.pallas_call(
        _proj_body,
        grid=(grid,),
        in_specs=[
            pl.BlockSpec((_ROW_BLK, d), lambda i: (i, 0)),
            pl.BlockSpec((d, d), lambda i: (0, 0)),
            pl.BlockSpec((1, d), lambda i: (0, 0)),
            pl.BlockSpec((d, d), lambda i: (0, 0)),
            pl.BlockSpec((1, d), lambda i: (0, 0)),
        ],
        out_specs=[
            pl.BlockSpec((_ROW_BLK, d), lambda i: (i, 0)),
            pl.BlockSpec((_ROW_BLK, d), lambda i: (i, 0)),
        ],
        out_shape=[
            jax.ShapeDtypeStruct((n_pad, d), jnp.float32),
            jax.ShapeDtypeStruct((n_pad, d), jnp.float32),
        ],
        interpret=interpret,
    )(side_p, Wk, bk[None, :], Wv, bv[None, :])

    try:
        info = plsc.get_sparse_core_info()
        num_cores, num_subcores = info.num_cores, info.num_subcores
    except Exception:
        num_cores, num_subcores = 2, 16
    num_workers = num_cores * num_subcores

    nh = 8
    rows_h = n_pad // nh
    rows_per_w = rows_h // num_workers
    mesh = plsc.VectorSubcoreMesh(core_axis_name="c", subcore_axis_name="s",
                                  num_cores=num_cores, num_subcores=num_subcores)
    sc_fn = pl.kernel(
        functools.partial(_sc_attend_body, n_pad, rows_per_w, num_cores),
        mesh=mesh,
        compiler_params=pltpu.CompilerParams(needs_layout_passes=False),
        interpret=interpret,
        out_type=jax.ShapeDtypeStruct((rows_h, d), jnp.float32),
        scratch_types=[
            pltpu.VMEM((rows_per_w * _K,), jnp.int32),
            pltpu.VMEM((2, _K, _SELW), jnp.float32),
            pltpu.VMEM((2, 16), jnp.int32),
            pltpu.VMEM((16,), jnp.int32),
            pltpu.VMEM((_K, d), jnp.float32),
            pltpu.VMEM((2, d), jnp.float32),
            pltpu.VMEM((32,), jnp.float32),
            pltpu.SemaphoreType.DMA,
            pltpu.SemaphoreType.DMA,
            pltpu.SemaphoreType.DMA,
        ],
    )

    outs = []
    for h in range(nh):
        ego_h = jax.lax.slice_in_dim(ego_p, h * rows_h, (h + 1) * rows_h, axis=0)
        smat_h, tmat_h = pl.pallas_call(
            functools.partial(_score_body, n, n_pad),
            grid=(rows_h // _ROW_BLK,),
            in_specs=[
                pl.BlockSpec((_ROW_BLK, d), lambda i: (i, 0)),
                pl.BlockSpec((d, d), lambda i: (0, 0)),
                pl.BlockSpec((1, d), lambda i: (0, 0)),
                pl.BlockSpec((n_pad, d), lambda i: (0, 0)),
            ],
            out_specs=[
                pl.BlockSpec((_ROW_BLK, n_pad), lambda i: (i, 0)),
                pl.BlockSpec((_ROW_BLK, _K), lambda i: (i, 0)),
            ],
            out_shape=[
                jax.ShapeDtypeStruct((rows_h, n_pad), jnp.float32),
                jax.ShapeDtypeStruct((rows_h, _K), jnp.int32),
            ],
            interpret=interpret,
        )(ego_h, Wq, bq[None, :], kmat)
        outs.append(sc_fn(smat_h.reshape(rows_h * nsel, _SELW),
                          tmat_h.reshape(rows_h * _K),
                          vmat))
    out = jnp.concatenate(outs, axis=0)
    return out[:n]


def kernel(ego_emb, side_emb, Wq, bq, Wk, bk, Wv, bv):
    return _run_hybrid(ego_emb, side_emb, Wq, bq, Wk, bk, Wv, bv)
